# baseline (speedup 1.0000x reference)
import functools

import jax
import jax.numpy as jnp
from jax import lax
from jax.experimental import pallas as pl
from jax.experimental.pallas import tpu as pltpu
from jax.experimental.pallas import tpu_sc as plsc

D_MODEL = 768
NODE_LEN = 512
BATCH = 16
CHUNK = 32
ZROWS = 16
RING = 4
LANES = 16
VECS_PER_ROW = D_MODEL // LANES
NWORKERS = 32
CPG = NODE_LEN // CHUNK
NCHUNK = BATCH * CPG // NWORKERS


def _mk_hidden_sc(flat, cu):
    total = flat.shape[0]
    mesh = plsc.VectorSubcoreMesh(core_axis_name="c", subcore_axis_name="s")

    @functools.partial(
        pl.kernel,
        out_type=(
            jax.ShapeDtypeStruct((BATCH, NODE_LEN, D_MODEL), jnp.float32),
            jax.ShapeDtypeStruct((BATCH, NODE_LEN), jnp.int32),
        ),
        mesh=mesh,
        compiler_params=pltpu.CompilerParams(needs_layout_passes=False),
        scratch_types=[
            pltpu.VMEM((LANES,), jnp.int32),
            pltpu.VMEM((LANES,), jnp.int32),
            pltpu.VMEM((RING, CHUNK), jnp.int32),
            pltpu.VMEM((CHUNK, D_MODEL), jnp.float32),
            pltpu.VMEM((CHUNK, D_MODEL), jnp.float32),
            pltpu.VMEM((CHUNK, D_MODEL), jnp.float32),
            pltpu.VMEM((CHUNK, D_MODEL), jnp.float32),
            pltpu.VMEM((ZROWS, D_MODEL), jnp.float32),
            pltpu.VMEM((8, NODE_LEN), jnp.int32),
            pltpu.SemaphoreType.DMA,
            pltpu.SemaphoreType.DMA,
            pltpu.SemaphoreType.DMA,
            pltpu.SemaphoreType.DMA,
            pltpu.SemaphoreType.DMA,
            pltpu.SemaphoreType.DMA,
            pltpu.SemaphoreType.DMA,
            pltpu.SemaphoreType.DMA,
            pltpu.SemaphoreType.DMA,
        ],
    )
    def k(flat_hbm, cu_hbm, out_hbm, mask_hbm,
          lo_v, hi_v, idx_v, buf0, buf1, buf2, buf3, zero_v, mask_v,
          si0, si1, si2, si3, so0, so1, so2, so3, sem_mask):
        bufs = (buf0, buf1, buf2, buf3)
        sems_in = (si0, si1, si2, si3)
        sems_out = (so0, so1, so2, so3)
        wid = lax.axis_index("s") * 2 + lax.axis_index("c")

        iota16 = lax.iota(jnp.int32, LANES)
        pltpu.async_copy(cu_hbm.at[pl.ds(0, LANES)], lo_v, si0)
        cp_hi = pltpu.async_copy(cu_hbm.at[pl.ds(8, 9)],
                                 hi_v.at[pl.ds(0, 9)], si1)
        pltpu.make_async_copy(cu_hbm.at[pl.ds(0, LANES)], lo_v, si0).wait()
        cp_hi.wait()
        lo = lo_v[...]
        hi = hi_v[...]

        def cu_at(i):
            from_lo = jnp.max(jnp.where(iota16 == i, lo, 0))
            from_hi = jnp.max(jnp.where(iota16 == i - 8, hi, 0))
            return jnp.where(i < LANES, from_lo, from_hi)

        def graph_len(gg):
            s = cu_at(gg)
            nn = cu_at(gg + 1) - s
            return s, jnp.where(nn == 1, 0, jnp.minimum(nn, NODE_LEN))

        G, C0, ST, V = [], [], [], []
        for kk in range(NCHUNK):
            t = wid + NWORKERS * kk
            gk = t // CPG
            c0k = (t % CPG) * CHUNK
            sk, lk = graph_len(gk)
            G.append(gk)
            C0.append(c0k)
            ST.append(sk)
            V.append(jnp.clip(lk - c0k, 0, CHUNK))

        zvec = jnp.zeros((LANES,), jnp.float32)

        def issue_in(c):
            b = c % RING

            @pl.when(V[c] > 0)
            def _():
                base = ST[c] + C0[c]
                for v in range(CHUNK // LANES):
                    idx_v[b, pl.ds(v * LANES, LANES)] = jnp.minimum(
                        iota16 + (base + v * LANES), total - 1)
                pltpu.async_copy(flat_hbm.at[idx_v.at[b]], bufs[b],
                                 sems_in[b])

        def absorb_out(c):
            pltpu.make_async_copy(
                bufs[c % RING],
                out_hbm.at[G[c], pl.ds(C0[c], CHUNK)],
                sems_out[c % RING]).wait()

        issue_in(0)
        issue_in(1)
        issue_in(2)

        @pl.when(wid < 2)
        def _():
            for gr in range(8):
                _, lg = graph_len(wid * 8 + gr)
                for v in range(NODE_LEN // LANES):
                    mask_v[gr, pl.ds(v * LANES, LANES)] = (
                        (iota16 + v * LANES) < lg).astype(jnp.int32)
            pltpu.async_copy(mask_v, mask_hbm.at[pl.ds(wid * 8, 8)], sem_mask)

        @pl.loop(0, ZROWS)
        def _(j):
            for kv in range(VECS_PER_ROW):
                zero_v[j, pl.ds(kv * LANES, LANES)] = zvec

        for c in range(NCHUNK):
            b = c % RING
            full = V[c] >= CHUNK
            empty = V[c] <= 0

            @pl.when(jnp.logical_not(empty))
            def _():
                pltpu.make_async_copy(flat_hbm.at[idx_v.at[b]], bufs[b],
                                      sems_in[b]).wait()

            @pl.when(jnp.logical_not(jnp.logical_or(full, empty)))
            def _():
                z0 = V[c]

                @pl.loop(0, CHUNK)
                def _(j):
                    @pl.when(j >= z0)
                    def _():
                        for kv in range(VECS_PER_ROW):
                            bufs[b][j, pl.ds(kv * LANES, LANES)] = zvec

            @pl.when(empty)
            def _():
                for z in range(CHUNK // ZROWS):
                    pltpu.async_copy(
                        zero_v,
                        out_hbm.at[G[c], pl.ds(C0[c] + z * ZROWS, ZROWS)],
                        sems_out[b])

            @pl.when(jnp.logical_not(empty))
            def _():
                pltpu.async_copy(
                    bufs[b], out_hbm.at[G[c], pl.ds(C0[c], CHUNK)],
                    sems_out[b])

            if c + RING - 1 < NCHUNK:
                if c >= 1:
                    absorb_out(c - 1)
                issue_in(c + RING - 1)

        for c in range(NCHUNK - RING, NCHUNK):
            absorb_out(c)

        @pl.when(wid < 2)
        def _():
            pltpu.make_async_copy(
                mask_v, mask_hbm.at[pl.ds(wid * 8, 8)], sem_mask).wait()

    return k(flat, cu)


def kernel(flat, cu_seqlens):
    return _mk_hidden_sc(flat, cu_seqlens)


# speedup vs baseline: 1.0228x; 1.0228x over previous
import functools

import jax
import jax.numpy as jnp
from jax import lax
from jax.experimental import pallas as pl
from jax.experimental.pallas import tpu as pltpu
from jax.experimental.pallas import tpu_sc as plsc

D_MODEL = 768
NODE_LEN = 512
BATCH = 16
CHUNK = 32
ZROWS = 16
RING = 4
HALF = NODE_LEN // 2
NCHUNK = HALF // CHUNK
LANES = 16
VECS_PER_ROW = D_MODEL // LANES


def _mk_hidden_sc(flat, cu):
    total = flat.shape[0]
    mesh = plsc.VectorSubcoreMesh(core_axis_name="c", subcore_axis_name="s")

    @functools.partial(
        pl.kernel,
        out_type=(
            jax.ShapeDtypeStruct((BATCH, NODE_LEN, D_MODEL), jnp.float32),
            jax.ShapeDtypeStruct((BATCH, NODE_LEN), jnp.int32),
        ),
        mesh=mesh,
        compiler_params=pltpu.CompilerParams(needs_layout_passes=False),
        scratch_types=[
            pltpu.VMEM((LANES,), jnp.int32),
            pltpu.VMEM((LANES,), jnp.int32),
            pltpu.VMEM((RING, CHUNK), jnp.int32),
            pltpu.VMEM((CHUNK, D_MODEL), jnp.float32),
            pltpu.VMEM((CHUNK, D_MODEL), jnp.float32),
            pltpu.VMEM((CHUNK, D_MODEL), jnp.float32),
            pltpu.VMEM((CHUNK, D_MODEL), jnp.float32),
            pltpu.VMEM((ZROWS, D_MODEL), jnp.float32),
            pltpu.VMEM((8, NODE_LEN), jnp.int32),
            pltpu.SemaphoreType.DMA,
            pltpu.SemaphoreType.DMA,
            pltpu.SemaphoreType.DMA,
            pltpu.SemaphoreType.DMA,
            pltpu.SemaphoreType.DMA,
            pltpu.SemaphoreType.DMA,
            pltpu.SemaphoreType.DMA,
            pltpu.SemaphoreType.DMA,
            pltpu.SemaphoreType.DMA,
        ],
    )
    def k(flat_hbm, cu_hbm, out_hbm, mask_hbm,
          lo_v, hi_v, idx_v, buf0, buf1, buf2, buf3, zero_v, mask_v,
          si0, si1, si2, si3, so0, so1, so2, so3, sem_mask):
        bufs = (buf0, buf1, buf2, buf3)
        sems_in = (si0, si1, si2, si3)
        sems_out = (so0, so1, so2, so3)
        wid = lax.axis_index("s") * 2 + lax.axis_index("c")
        g = wid // 2
        r0 = (wid % 2) * HALF

        iota16 = lax.iota(jnp.int32, LANES)
        pltpu.async_copy(cu_hbm.at[pl.ds(0, LANES)], lo_v, si0)
        cp_hi = pltpu.async_copy(cu_hbm.at[pl.ds(8, 9)],
                                 hi_v.at[pl.ds(0, 9)], si1)
        pltpu.make_async_copy(cu_hbm.at[pl.ds(0, LANES)], lo_v, si0).wait()
        cp_hi.wait()
        lo = lo_v[...]
        hi = hi_v[...]

        def cu_at(i):
            from_lo = jnp.max(jnp.where(iota16 == i, lo, 0))
            from_hi = jnp.max(jnp.where(iota16 == i - 8, hi, 0))
            return jnp.where(i < LANES, from_lo, from_hi)

        def graph_len(gg):
            s = cu_at(gg)
            nn = cu_at(gg + 1) - s
            return s, jnp.where(nn == 1, 0, jnp.minimum(nn, NODE_LEN))

        start, L = graph_len(g)
        s_rel = jnp.clip(L - r0, 0, HALF)

        zvec = jnp.zeros((LANES,), jnp.float32)

        def issue_in(c):
            b = c % RING

            @pl.when(s_rel > c * CHUNK)
            def _():
                base = start + r0 + c * CHUNK
                for v in range(CHUNK // LANES):
                    idx_v[b, pl.ds(v * LANES, LANES)] = jnp.minimum(
                        iota16 + (base + v * LANES), total - 1)
                pltpu.async_copy(flat_hbm.at[idx_v.at[b]], bufs[b],
                                 sems_in[b])

        def absorb_out(c):
            pltpu.make_async_copy(
                bufs[c % RING], out_hbm.at[g, pl.ds(r0 + c * CHUNK, CHUNK)],
                sems_out[c % RING]).wait()

        issue_in(0)
        issue_in(1)
        issue_in(2)

        @pl.when(wid < 2)
        def _():
            for gr in range(8):
                _, lg = graph_len(wid * 8 + gr)
                for v in range(NODE_LEN // LANES):
                    mask_v[gr, pl.ds(v * LANES, LANES)] = (
                        (iota16 + v * LANES) < lg).astype(jnp.int32)
            pltpu.async_copy(mask_v, mask_hbm.at[pl.ds(wid * 8, 8)], sem_mask)

        @pl.when(s_rel < HALF)
        def _():
            @pl.loop(0, ZROWS)
            def _(j):
                for kv in range(VECS_PER_ROW):
                    zero_v[j, pl.ds(kv * LANES, LANES)] = zvec

        for c in range(NCHUNK):
            c0 = c * CHUNK
            b = c % RING
            full = s_rel >= (c0 + CHUNK)
            empty = s_rel <= c0

            @pl.when(jnp.logical_not(empty))
            def _():
                pltpu.make_async_copy(flat_hbm.at[idx_v.at[b]], bufs[b],
                                      sems_in[b]).wait()

            @pl.when(jnp.logical_not(jnp.logical_or(full, empty)))
            def _():
                z0 = s_rel - c0

                @pl.loop(0, CHUNK)
                def _(j):
                    @pl.when(j >= z0)
                    def _():
                        for kv in range(VECS_PER_ROW):
                            bufs[b][j, pl.ds(kv * LANES, LANES)] = zvec

            @pl.when(empty)
            def _():
                for z in range(CHUNK // ZROWS):
                    pltpu.async_copy(
                        zero_v,
                        out_hbm.at[g, pl.ds(r0 + c0 + z * ZROWS, ZROWS)],
                        sems_out[b])

            @pl.when(jnp.logical_not(empty))
            def _():
                pltpu.async_copy(
                    bufs[b], out_hbm.at[g, pl.ds(r0 + c0, CHUNK)], sems_out[b])

            if c + RING - 1 < NCHUNK:
                if c >= 1:
                    absorb_out(c - 1)
                issue_in(c + RING - 1)

        for c in range(NCHUNK - RING, NCHUNK):
            absorb_out(c)

        @pl.when(wid < 2)
        def _():
            pltpu.make_async_copy(
                mask_v, mask_hbm.at[pl.ds(wid * 8, 8)], sem_mask).wait()

    return k(flat, cu)


def kernel(flat, cu_seqlens):
    return _mk_hidden_sc(flat, cu_seqlens)
